# Initial kernel scaffold; baseline (speedup 1.0000x reference)
import jax
import jax.numpy as jnp
from jax.experimental import pallas as pl


def kernel(x, edge_index, W1, b1, W2, b2, W3, b3, g1, be1, g2, be2):
    raise NotImplementedError("write your pallas kernel here")


# speedup vs baseline: 5.8527x; 5.8527x over previous
import functools

import jax
import jax.numpy as jnp
from jax import lax
from jax.experimental import pallas as pl
from jax.experimental.pallas import tpu as pltpu
from jax.experimental.pallas import tpu_sc as plsc

N = 10000
E = 320000
D = 128
NC, NS = 2, 16
NW = NC * NS
EB = 1024
JB = EB // 128
SUB = 2
WAVES = JB // SUB
BPT = 10
EP = NW * BPT * EB
OROWS = 632
ACC_ROWS = 10240
YPAD = N + 8
RB = 1000
GRID = N // RB
EPS = 1e-5

_mesh = plsc.VectorSubcoreMesh(core_axis_name="c", subcore_axis_name="s")


@functools.partial(
    pl.kernel,
    out_type=jax.ShapeDtypeStruct((NC, N, D), jnp.float32),
    mesh=_mesh,
    scratch_types=[
        pltpu.VMEM((JB, 128), jnp.int32),
        pltpu.VMEM((JB, 128), jnp.int32),
        pltpu.VMEM((SUB * 128, D), jnp.float32),
        pltpu.VMEM_SHARED((ACC_ROWS, D), jnp.float32),
        pltpu.SemaphoreType.DMA,
    ],
)
def _sc_agg(y_hbm, src_hbm, dst_hbm, zeros_hbm, out_hbm,
            src_v, dst_v, rows_v, acc, sem):
    cid = lax.axis_index("c")
    sid = lax.axis_index("s")
    wid = sid * NC + cid
    pltpu.sync_copy(zeros_hbm, acc.at[pl.ds(sid * (ACC_ROWS // NS), ACC_ROWS // NS)])
    plsc.subcore_barrier()

    def body(b, carry):
        g = wid * BPT + b
        pltpu.sync_copy(src_hbm.at[pl.ds(g * JB, JB)], src_v)
        pltpu.sync_copy(dst_hbm.at[pl.ds(g * JB, JB)], dst_v)
        for h in range(WAVES):
            descs = [
                pltpu.async_copy(y_hbm.at[src_v.at[h * SUB + j]],
                                 rows_v.at[pl.ds(j * 128, 128)], sem)
                for j in range(SUB)
            ]
            for d in descs:
                d.wait()
            for j in range(SUB):
                pltpu.sync_copy(rows_v.at[pl.ds(j * 128, 128)],
                                acc.at[dst_v.at[h * SUB + j]], add=True)
        return carry

    lax.fori_loop(0, BPT, body, 0)
    plsc.subcore_barrier()

    @pl.when(sid < NS - 1)
    def _():
        pltpu.sync_copy(acc.at[pl.ds(sid * OROWS, OROWS)],
                        out_hbm.at[cid, pl.ds(sid * OROWS, OROWS)])

    @pl.when(sid == NS - 1)
    def _():
        pltpu.sync_copy(acc.at[pl.ds((NS - 1) * OROWS, N - (NS - 1) * OROWS)],
                        out_hbm.at[cid, pl.ds((NS - 1) * OROWS, N - (NS - 1) * OROWS)])


def _dinv_body(dp_ref, o_ref):
    deg = 1.0 + dp_ref[0, :, 0:1] + dp_ref[1, :, 0:1]
    o_ref[...] = lax.rsqrt(deg)


def _dinv(degp):
    return pl.pallas_call(
        _dinv_body,
        grid=(GRID,),
        in_specs=[pl.BlockSpec((NC, RB, D), lambda i: (0, i, 0))],
        out_specs=pl.BlockSpec((RB, 1), lambda i: (i, 0)),
        out_shape=jax.ShapeDtypeStruct((N, 1), jnp.float32),
    )(degp)


def _mm_body(z_ref, w_ref, dinv_ref, o_ref):
    h = jnp.dot(z_ref[...], w_ref[...], preferred_element_type=jnp.float32)
    o_ref[...] = h * dinv_ref[...]


def _mm_scale(z, W, dinv):
    return pl.pallas_call(
        _mm_body,
        grid=(GRID,),
        in_specs=[
            pl.BlockSpec((RB, D), lambda i: (i, 0)),
            pl.BlockSpec((D, D), lambda i: (0, 0)),
            pl.BlockSpec((RB, 1), lambda i: (i, 0)),
        ],
        out_specs=pl.BlockSpec((RB, D), lambda i: (i, 0)),
        out_shape=jax.ShapeDtypeStruct((N, D), jnp.float32),
    )(z, W, dinv)


def _comb_body(p_ref, y_ref, dinv_ref, b_ref, o_ref, ps_ref, pq_ref,
               acc_s, acc_q):
    i = pl.program_id(0)
    o = dinv_ref[...] * (p_ref[0] + p_ref[1] + y_ref[...]) + b_ref[...]
    o_ref[...] = o
    s = jnp.sum(o, axis=0, keepdims=True)
    q = jnp.sum(o * o, axis=0, keepdims=True)

    @pl.when(i == 0)
    def _():
        acc_s[...] = s
        acc_q[...] = q

    @pl.when(i > 0)
    def _():
        acc_s[...] += s
        acc_q[...] += q

    @pl.when(i == GRID - 1)
    def _():
        ps_ref[...] = acc_s[...]
        pq_ref[...] = acc_q[...]


def _combine_stats(P, y, dinv, b):
    return pl.pallas_call(
        _comb_body,
        grid=(GRID,),
        in_specs=[
            pl.BlockSpec((NC, RB, D), lambda i: (0, i, 0)),
            pl.BlockSpec((RB, D), lambda i: (i, 0)),
            pl.BlockSpec((RB, 1), lambda i: (i, 0)),
            pl.BlockSpec((1, D), lambda i: (0, 0)),
        ],
        out_specs=[
            pl.BlockSpec((RB, D), lambda i: (i, 0)),
            pl.BlockSpec((1, D), lambda i: (0, 0)),
            pl.BlockSpec((1, D), lambda i: (0, 0)),
        ],
        out_shape=[
            jax.ShapeDtypeStruct((N, D), jnp.float32),
            jax.ShapeDtypeStruct((1, D), jnp.float32),
            jax.ShapeDtypeStruct((1, D), jnp.float32),
        ],
        scratch_shapes=[
            pltpu.VMEM((1, D), jnp.float32),
            pltpu.VMEM((1, D), jnp.float32),
        ],
    )(P, y, dinv, b)


def _bn_mm_body(o_ref, ps_ref, pq_ref, g_ref, be_ref, w_ref, dinv_ref, y_ref):
    mu = ps_ref[...] * (1.0 / N)
    var = pq_ref[...] * (1.0 / N) - mu * mu
    scale = lax.rsqrt(var + EPS) * g_ref[...]
    z = jnp.maximum((o_ref[...] - mu) * scale + be_ref[...], 0.0)
    h = jnp.dot(z, w_ref[...], preferred_element_type=jnp.float32)
    y_ref[...] = h * dinv_ref[...]


def _bn_relu_mm(o, ps, pq, g, be, W, dinv):
    return pl.pallas_call(
        _bn_mm_body,
        grid=(GRID,),
        in_specs=[
            pl.BlockSpec((RB, D), lambda i: (i, 0)),
            pl.BlockSpec((1, D), lambda i: (0, 0)),
            pl.BlockSpec((1, D), lambda i: (0, 0)),
            pl.BlockSpec((1, D), lambda i: (0, 0)),
            pl.BlockSpec((1, D), lambda i: (0, 0)),
            pl.BlockSpec((D, D), lambda i: (0, 0)),
            pl.BlockSpec((RB, 1), lambda i: (i, 0)),
        ],
        out_specs=pl.BlockSpec((RB, D), lambda i: (i, 0)),
        out_shape=jax.ShapeDtypeStruct((N, D), jnp.float32),
    )(o, ps, pq, g, be, W, dinv)


def _final_body(p_ref, y_ref, dinv_ref, b_ref, o_ref):
    o = dinv_ref[...] * (p_ref[0] + p_ref[1] + y_ref[...]) + b_ref[...]
    o_ref[...] = jnp.maximum(o, 0.0)


def _final(P, y, dinv, b):
    return pl.pallas_call(
        _final_body,
        grid=(GRID,),
        in_specs=[
            pl.BlockSpec((NC, RB, D), lambda i: (0, i, 0)),
            pl.BlockSpec((RB, D), lambda i: (i, 0)),
            pl.BlockSpec((RB, 1), lambda i: (i, 0)),
            pl.BlockSpec((1, D), lambda i: (0, 0)),
        ],
        out_specs=pl.BlockSpec((RB, D), lambda i: (i, 0)),
        out_shape=jax.ShapeDtypeStruct((N, D), jnp.float32),
    )(P, y, dinv, b)


def _pad_y(y):
    return jnp.concatenate([y, jnp.zeros((YPAD - N, D), jnp.float32)], axis=0)


def kernel(x, edge_index, W1, b1, W2, b2, W3, b3, g1, be1, g2, be2):
    src = edge_index[0].astype(jnp.int32)
    dst = edge_index[1].astype(jnp.int32)
    pad = jnp.full((EP - E,), N, jnp.int32)
    srcr = jnp.concatenate([src, pad]).reshape(EP // 128, 128)
    dstr = jnp.concatenate([dst, pad]).reshape(EP // 128, 128)
    z128 = jnp.zeros((ACC_ROWS // NS, D), jnp.float32)
    b1r, b2r, b3r = b1.reshape(1, D), b2.reshape(1, D), b3.reshape(1, D)
    g1r, be1r = g1.reshape(1, D), be1.reshape(1, D)
    g2r, be2r = g2.reshape(1, D), be2.reshape(1, D)

    ones_tab = jnp.concatenate(
        [jnp.ones((N, D), jnp.float32), jnp.zeros((YPAD - N, D), jnp.float32)])
    degp = _sc_agg(ones_tab, srcr, dstr, z128)
    dinv = _dinv(degp)

    y1 = _mm_scale(x, W1, dinv)
    P1 = _sc_agg(_pad_y(y1), srcr, dstr, z128)
    o1, ps1, pq1 = _combine_stats(P1, y1, dinv, b1r)
    y2 = _bn_relu_mm(o1, ps1, pq1, g1r, be1r, W2, dinv)
    P2 = _sc_agg(_pad_y(y2), srcr, dstr, z128)
    o2, ps2, pq2 = _combine_stats(P2, y2, dinv, b2r)
    y3 = _bn_relu_mm(o2, ps2, pq2, g2r, be2r, W3, dinv)
    P3 = _sc_agg(_pad_y(y3), srcr, dstr, z128)
    return _final(P3, y3, dinv, b3r)


# speedup vs baseline: 6.1605x; 1.0526x over previous
import functools

import jax
import jax.numpy as jnp
from jax import lax
from jax.experimental import pallas as pl
from jax.experimental.pallas import tpu as pltpu
from jax.experimental.pallas import tpu_sc as plsc

N = 10000
E = 320000
D = 128
NC, NS = 2, 16
NW = NC * NS
EB = 1024
JB = EB // 128
SUB = 2
WAVES = JB // SUB
BPT = 10
EP = NW * BPT * EB
OROWS = 632
ACC_ROWS = 10240
YPAD = N + 8
RB = 1000
GRID = N // RB
EPS = 1e-5

_mesh = plsc.VectorSubcoreMesh(core_axis_name="c", subcore_axis_name="s")


@functools.partial(
    pl.kernel,
    out_type=jax.ShapeDtypeStruct((NC, N, D), jnp.float32),
    mesh=_mesh,
    scratch_types=[
        pltpu.VMEM((JB, 128), jnp.int32),
        pltpu.VMEM((JB, 128), jnp.int32),
        pltpu.VMEM((128, D), jnp.float32),
        pltpu.VMEM((128, D), jnp.float32),
        pltpu.VMEM_SHARED((ACC_ROWS, D), jnp.float32),
        pltpu.SemaphoreType.DMA,
        pltpu.SemaphoreType.DMA,
    ],
)
def _sc_agg(y_hbm, src_hbm, dst_hbm, zeros_hbm, out_hbm,
            src_v, dst_v, rows_a, rows_b, acc, sem_g, sem_s):
    cid = lax.axis_index("c")
    sid = lax.axis_index("s")
    wid = sid * NC + cid
    bufs = (rows_a, rows_b)
    pltpu.sync_copy(zeros_hbm, acc.at[pl.ds(sid * (ACC_ROWS // NS), ACC_ROWS // NS)])
    plsc.subcore_barrier()

    def body(b, carry):
        g = wid * BPT + b
        pltpu.sync_copy(src_hbm.at[pl.ds(g * JB, JB)], src_v)
        pltpu.sync_copy(dst_hbm.at[pl.ds(g * JB, JB)], dst_v)
        gat = [None] * JB
        sca = [None] * JB
        gat[0] = pltpu.async_copy(y_hbm.at[src_v.at[0]], bufs[0], sem_g)
        for j in range(JB):
            buf = bufs[j % 2]
            gat[j].wait()
            sca[j] = pltpu.async_copy(buf, acc.at[dst_v.at[j]], sem_s, add=True)
            if j + 1 < JB:
                if j >= 1:
                    sca[j - 1].wait()
                gat[j + 1] = pltpu.async_copy(y_hbm.at[src_v.at[j + 1]],
                                              bufs[(j + 1) % 2], sem_g)
        sca[JB - 2].wait()
        sca[JB - 1].wait()
        return carry

    lax.fori_loop(0, BPT, body, 0)
    plsc.subcore_barrier()

    @pl.when(sid < NS - 1)
    def _():
        pltpu.sync_copy(acc.at[pl.ds(sid * OROWS, OROWS)],
                        out_hbm.at[cid, pl.ds(sid * OROWS, OROWS)])

    @pl.when(sid == NS - 1)
    def _():
        pltpu.sync_copy(acc.at[pl.ds((NS - 1) * OROWS, N - (NS - 1) * OROWS)],
                        out_hbm.at[cid, pl.ds((NS - 1) * OROWS, N - (NS - 1) * OROWS)])


def _dinv_body(dp_ref, o_ref):
    deg = 1.0 + dp_ref[0, :, 0:1] + dp_ref[1, :, 0:1]
    o_ref[...] = lax.rsqrt(deg)


def _dinv(degp):
    return pl.pallas_call(
        _dinv_body,
        grid=(GRID,),
        in_specs=[pl.BlockSpec((NC, RB, D), lambda i: (0, i, 0))],
        out_specs=pl.BlockSpec((RB, 1), lambda i: (i, 0)),
        out_shape=jax.ShapeDtypeStruct((N, 1), jnp.float32),
    )(degp)


def _mm_body(z_ref, w_ref, dinv_ref, o_ref):
    h = jnp.dot(z_ref[...], w_ref[...], preferred_element_type=jnp.float32)
    o_ref[...] = h * dinv_ref[...]


def _mm_scale(z, W, dinv):
    return pl.pallas_call(
        _mm_body,
        grid=(GRID,),
        in_specs=[
            pl.BlockSpec((RB, D), lambda i: (i, 0)),
            pl.BlockSpec((D, D), lambda i: (0, 0)),
            pl.BlockSpec((RB, 1), lambda i: (i, 0)),
        ],
        out_specs=pl.BlockSpec((RB, D), lambda i: (i, 0)),
        out_shape=jax.ShapeDtypeStruct((N, D), jnp.float32),
    )(z, W, dinv)


def _comb_body(p_ref, y_ref, dinv_ref, b_ref, o_ref, ps_ref, pq_ref,
               acc_s, acc_q):
    i = pl.program_id(0)
    o = dinv_ref[...] * (p_ref[0] + p_ref[1] + y_ref[...]) + b_ref[...]
    o_ref[...] = o
    s = jnp.sum(o, axis=0, keepdims=True)
    q = jnp.sum(o * o, axis=0, keepdims=True)

    @pl.when(i == 0)
    def _():
        acc_s[...] = s
        acc_q[...] = q

    @pl.when(i > 0)
    def _():
        acc_s[...] += s
        acc_q[...] += q

    @pl.when(i == GRID - 1)
    def _():
        ps_ref[...] = acc_s[...]
        pq_ref[...] = acc_q[...]


def _combine_stats(P, y, dinv, b):
    return pl.pallas_call(
        _comb_body,
        grid=(GRID,),
        in_specs=[
            pl.BlockSpec((NC, RB, D), lambda i: (0, i, 0)),
            pl.BlockSpec((RB, D), lambda i: (i, 0)),
            pl.BlockSpec((RB, 1), lambda i: (i, 0)),
            pl.BlockSpec((1, D), lambda i: (0, 0)),
        ],
        out_specs=[
            pl.BlockSpec((RB, D), lambda i: (i, 0)),
            pl.BlockSpec((1, D), lambda i: (0, 0)),
            pl.BlockSpec((1, D), lambda i: (0, 0)),
        ],
        out_shape=[
            jax.ShapeDtypeStruct((N, D), jnp.float32),
            jax.ShapeDtypeStruct((1, D), jnp.float32),
            jax.ShapeDtypeStruct((1, D), jnp.float32),
        ],
        scratch_shapes=[
            pltpu.VMEM((1, D), jnp.float32),
            pltpu.VMEM((1, D), jnp.float32),
        ],
    )(P, y, dinv, b)


def _bn_mm_body(o_ref, ps_ref, pq_ref, g_ref, be_ref, w_ref, dinv_ref, y_ref):
    mu = ps_ref[...] * (1.0 / N)
    var = pq_ref[...] * (1.0 / N) - mu * mu
    scale = lax.rsqrt(var + EPS) * g_ref[...]
    z = jnp.maximum((o_ref[...] - mu) * scale + be_ref[...], 0.0)
    h = jnp.dot(z, w_ref[...], preferred_element_type=jnp.float32)
    y_ref[...] = h * dinv_ref[...]


def _bn_relu_mm(o, ps, pq, g, be, W, dinv):
    return pl.pallas_call(
        _bn_mm_body,
        grid=(GRID,),
        in_specs=[
            pl.BlockSpec((RB, D), lambda i: (i, 0)),
            pl.BlockSpec((1, D), lambda i: (0, 0)),
            pl.BlockSpec((1, D), lambda i: (0, 0)),
            pl.BlockSpec((1, D), lambda i: (0, 0)),
            pl.BlockSpec((1, D), lambda i: (0, 0)),
            pl.BlockSpec((D, D), lambda i: (0, 0)),
            pl.BlockSpec((RB, 1), lambda i: (i, 0)),
        ],
        out_specs=pl.BlockSpec((RB, D), lambda i: (i, 0)),
        out_shape=jax.ShapeDtypeStruct((N, D), jnp.float32),
    )(o, ps, pq, g, be, W, dinv)


def _final_body(p_ref, y_ref, dinv_ref, b_ref, o_ref):
    o = dinv_ref[...] * (p_ref[0] + p_ref[1] + y_ref[...]) + b_ref[...]
    o_ref[...] = jnp.maximum(o, 0.0)


def _final(P, y, dinv, b):
    return pl.pallas_call(
        _final_body,
        grid=(GRID,),
        in_specs=[
            pl.BlockSpec((NC, RB, D), lambda i: (0, i, 0)),
            pl.BlockSpec((RB, D), lambda i: (i, 0)),
            pl.BlockSpec((RB, 1), lambda i: (i, 0)),
            pl.BlockSpec((1, D), lambda i: (0, 0)),
        ],
        out_specs=pl.BlockSpec((RB, D), lambda i: (i, 0)),
        out_shape=jax.ShapeDtypeStruct((N, D), jnp.float32),
    )(P, y, dinv, b)


def _pad_y(y):
    return jnp.concatenate([y, jnp.zeros((YPAD - N, D), jnp.float32)], axis=0)


def kernel(x, edge_index, W1, b1, W2, b2, W3, b3, g1, be1, g2, be2):
    src = edge_index[0].astype(jnp.int32)
    dst = edge_index[1].astype(jnp.int32)
    pad = jnp.full((EP - E,), N, jnp.int32)
    srcr = jnp.concatenate([src, pad]).reshape(EP // 128, 128)
    dstr = jnp.concatenate([dst, pad]).reshape(EP // 128, 128)
    z128 = jnp.zeros((ACC_ROWS // NS, D), jnp.float32)
    b1r, b2r, b3r = b1.reshape(1, D), b2.reshape(1, D), b3.reshape(1, D)
    g1r, be1r = g1.reshape(1, D), be1.reshape(1, D)
    g2r, be2r = g2.reshape(1, D), be2.reshape(1, D)

    ones_tab = jnp.concatenate(
        [jnp.ones((N, D), jnp.float32), jnp.zeros((YPAD - N, D), jnp.float32)])
    degp = _sc_agg(ones_tab, srcr, dstr, z128)
    dinv = _dinv(degp)

    y1 = _mm_scale(x, W1, dinv)
    P1 = _sc_agg(_pad_y(y1), srcr, dstr, z128)
    o1, ps1, pq1 = _combine_stats(P1, y1, dinv, b1r)
    y2 = _bn_relu_mm(o1, ps1, pq1, g1r, be1r, W2, dinv)
    P2 = _sc_agg(_pad_y(y2), srcr, dstr, z128)
    o2, ps2, pq2 = _combine_stats(P2, y2, dinv, b2r)
    y3 = _bn_relu_mm(o2, ps2, pq2, g2r, be2r, W3, dinv)
    P3 = _sc_agg(_pad_y(y3), srcr, dstr, z128)
    return _final(P3, y3, dinv, b3r)


# speedup vs baseline: 6.2317x; 1.0116x over previous
import functools

import jax
import jax.numpy as jnp
from jax import lax
from jax.experimental import pallas as pl
from jax.experimental.pallas import tpu as pltpu
from jax.experimental.pallas import tpu_sc as plsc

N = 10000
E = 320000
D = 128
NC, NS = 2, 16
NW = NC * NS
EB = 1024
JB = EB // 128
SUB = 2
WAVES = JB // SUB
BPT = 10
EP = NW * BPT * EB
OROWS = 632
ACC_ROWS = 10240
YPAD = N + 8
RB = 1000
GRID = N // RB
EPS = 1e-5

_mesh = plsc.VectorSubcoreMesh(core_axis_name="c", subcore_axis_name="s")


@functools.partial(
    pl.kernel,
    out_type=jax.ShapeDtypeStruct((NC, N, D), jnp.float32),
    mesh=_mesh,
    scratch_types=[
        pltpu.VMEM((JB, 128), jnp.int32),
        pltpu.VMEM((JB, 128), jnp.int32),
        pltpu.VMEM((128, D), jnp.float32),
        pltpu.VMEM((128, D), jnp.float32),
        pltpu.VMEM_SHARED((ACC_ROWS, D), jnp.float32),
        pltpu.SemaphoreType.DMA,
        pltpu.SemaphoreType.DMA,
    ],
)
def _sc_agg(y_hbm, src_hbm, dst_hbm, zeros_hbm, out_hbm,
            src_v, dst_v, rows_a, rows_b, acc, sem_g, sem_s):
    cid = lax.axis_index("c")
    sid = lax.axis_index("s")
    wid = sid * NC + cid
    bufs = (rows_a, rows_b)
    pltpu.sync_copy(zeros_hbm, acc.at[pl.ds(sid * (ACC_ROWS // NS), ACC_ROWS // NS)])
    plsc.subcore_barrier()

    def body(b, carry):
        g = wid * BPT + b
        pltpu.sync_copy(src_hbm.at[pl.ds(g * JB, JB)], src_v)
        pltpu.sync_copy(dst_hbm.at[pl.ds(g * JB, JB)], dst_v)
        gat = [None] * JB
        sca = [None] * JB
        gat[0] = pltpu.async_copy(y_hbm.at[src_v.at[0]], bufs[0], sem_g)
        for j in range(JB):
            buf = bufs[j % 2]
            gat[j].wait()
            sca[j] = pltpu.async_copy(buf, acc.at[dst_v.at[j]], sem_s, add=True)
            if j + 1 < JB:
                if j >= 1:
                    sca[j - 1].wait()
                gat[j + 1] = pltpu.async_copy(y_hbm.at[src_v.at[j + 1]],
                                              bufs[(j + 1) % 2], sem_g)
        sca[JB - 2].wait()
        sca[JB - 1].wait()
        return carry

    lax.fori_loop(0, BPT, body, 0)
    plsc.subcore_barrier()

    @pl.when(sid < NS - 1)
    def _():
        pltpu.sync_copy(acc.at[pl.ds(sid * OROWS, OROWS)],
                        out_hbm.at[cid, pl.ds(sid * OROWS, OROWS)])

    @pl.when(sid == NS - 1)
    def _():
        pltpu.sync_copy(acc.at[pl.ds((NS - 1) * OROWS, N - (NS - 1) * OROWS)],
                        out_hbm.at[cid, pl.ds((NS - 1) * OROWS, N - (NS - 1) * OROWS)])


@functools.partial(
    pl.kernel,
    out_type=jax.ShapeDtypeStruct((NC, N, D), jnp.float32),
    mesh=_mesh,
    scratch_types=[
        pltpu.VMEM((JB, 128), jnp.int32),
        pltpu.VMEM((128, D), jnp.float32),
        pltpu.VMEM_SHARED((ACC_ROWS, D), jnp.float32),
        pltpu.SemaphoreType.DMA,
    ],
)
def _sc_deg(dst_hbm, ones_hbm, zeros_hbm, out_hbm, dst_v, ones_v, acc, sem_s):
    cid = lax.axis_index("c")
    sid = lax.axis_index("s")
    wid = sid * NC + cid
    pltpu.sync_copy(ones_hbm, ones_v)
    pltpu.sync_copy(zeros_hbm, acc.at[pl.ds(sid * (ACC_ROWS // NS), ACC_ROWS // NS)])
    plsc.subcore_barrier()

    def body(b, carry):
        g = wid * BPT + b
        pltpu.sync_copy(dst_hbm.at[pl.ds(g * JB, JB)], dst_v)
        sca = [
            pltpu.async_copy(ones_v, acc.at[dst_v.at[j]], sem_s, add=True)
            for j in range(JB)
        ]
        for s in sca:
            s.wait()
        return carry

    lax.fori_loop(0, BPT, body, 0)
    plsc.subcore_barrier()

    @pl.when(sid < NS - 1)
    def _():
        pltpu.sync_copy(acc.at[pl.ds(sid * OROWS, OROWS)],
                        out_hbm.at[cid, pl.ds(sid * OROWS, OROWS)])

    @pl.when(sid == NS - 1)
    def _():
        pltpu.sync_copy(acc.at[pl.ds((NS - 1) * OROWS, N - (NS - 1) * OROWS)],
                        out_hbm.at[cid, pl.ds((NS - 1) * OROWS, N - (NS - 1) * OROWS)])


def _dinv_body(dp_ref, o_ref):
    deg = 1.0 + dp_ref[0, :, 0:1] + dp_ref[1, :, 0:1]
    o_ref[...] = lax.rsqrt(deg)


def _dinv(degp):
    return pl.pallas_call(
        _dinv_body,
        grid=(GRID,),
        in_specs=[pl.BlockSpec((NC, RB, D), lambda i: (0, i, 0))],
        out_specs=pl.BlockSpec((RB, 1), lambda i: (i, 0)),
        out_shape=jax.ShapeDtypeStruct((N, 1), jnp.float32),
    )(degp)


def _mm_body(z_ref, w_ref, dinv_ref, o_ref):
    h = jnp.dot(z_ref[...], w_ref[...], preferred_element_type=jnp.float32)
    o_ref[...] = h * dinv_ref[...]


def _mm_scale(z, W, dinv):
    return pl.pallas_call(
        _mm_body,
        grid=(GRID,),
        in_specs=[
            pl.BlockSpec((RB, D), lambda i: (i, 0)),
            pl.BlockSpec((D, D), lambda i: (0, 0)),
            pl.BlockSpec((RB, 1), lambda i: (i, 0)),
        ],
        out_specs=pl.BlockSpec((RB, D), lambda i: (i, 0)),
        out_shape=jax.ShapeDtypeStruct((N, D), jnp.float32),
    )(z, W, dinv)


def _comb_body(p_ref, y_ref, dinv_ref, b_ref, o_ref, ps_ref, pq_ref,
               acc_s, acc_q):
    i = pl.program_id(0)
    o = dinv_ref[...] * (p_ref[0] + p_ref[1] + y_ref[...]) + b_ref[...]
    o_ref[...] = o
    s = jnp.sum(o, axis=0, keepdims=True)
    q = jnp.sum(o * o, axis=0, keepdims=True)

    @pl.when(i == 0)
    def _():
        acc_s[...] = s
        acc_q[...] = q

    @pl.when(i > 0)
    def _():
        acc_s[...] += s
        acc_q[...] += q

    @pl.when(i == GRID - 1)
    def _():
        ps_ref[...] = acc_s[...]
        pq_ref[...] = acc_q[...]


def _combine_stats(P, y, dinv, b):
    return pl.pallas_call(
        _comb_body,
        grid=(GRID,),
        in_specs=[
            pl.BlockSpec((NC, RB, D), lambda i: (0, i, 0)),
            pl.BlockSpec((RB, D), lambda i: (i, 0)),
            pl.BlockSpec((RB, 1), lambda i: (i, 0)),
            pl.BlockSpec((1, D), lambda i: (0, 0)),
        ],
        out_specs=[
            pl.BlockSpec((RB, D), lambda i: (i, 0)),
            pl.BlockSpec((1, D), lambda i: (0, 0)),
            pl.BlockSpec((1, D), lambda i: (0, 0)),
        ],
        out_shape=[
            jax.ShapeDtypeStruct((N, D), jnp.float32),
            jax.ShapeDtypeStruct((1, D), jnp.float32),
            jax.ShapeDtypeStruct((1, D), jnp.float32),
        ],
        scratch_shapes=[
            pltpu.VMEM((1, D), jnp.float32),
            pltpu.VMEM((1, D), jnp.float32),
        ],
    )(P, y, dinv, b)


def _bn_mm_body(o_ref, ps_ref, pq_ref, g_ref, be_ref, w_ref, dinv_ref, y_ref):
    mu = ps_ref[...] * (1.0 / N)
    var = pq_ref[...] * (1.0 / N) - mu * mu
    scale = lax.rsqrt(var + EPS) * g_ref[...]
    z = jnp.maximum((o_ref[...] - mu) * scale + be_ref[...], 0.0)
    h = jnp.dot(z, w_ref[...], preferred_element_type=jnp.float32)
    y_ref[...] = h * dinv_ref[...]


def _bn_relu_mm(o, ps, pq, g, be, W, dinv):
    return pl.pallas_call(
        _bn_mm_body,
        grid=(GRID,),
        in_specs=[
            pl.BlockSpec((RB, D), lambda i: (i, 0)),
            pl.BlockSpec((1, D), lambda i: (0, 0)),
            pl.BlockSpec((1, D), lambda i: (0, 0)),
            pl.BlockSpec((1, D), lambda i: (0, 0)),
            pl.BlockSpec((1, D), lambda i: (0, 0)),
            pl.BlockSpec((D, D), lambda i: (0, 0)),
            pl.BlockSpec((RB, 1), lambda i: (i, 0)),
        ],
        out_specs=pl.BlockSpec((RB, D), lambda i: (i, 0)),
        out_shape=jax.ShapeDtypeStruct((N, D), jnp.float32),
    )(o, ps, pq, g, be, W, dinv)


def _final_body(p_ref, y_ref, dinv_ref, b_ref, o_ref):
    o = dinv_ref[...] * (p_ref[0] + p_ref[1] + y_ref[...]) + b_ref[...]
    o_ref[...] = jnp.maximum(o, 0.0)


def _final(P, y, dinv, b):
    return pl.pallas_call(
        _final_body,
        grid=(GRID,),
        in_specs=[
            pl.BlockSpec((NC, RB, D), lambda i: (0, i, 0)),
            pl.BlockSpec((RB, D), lambda i: (i, 0)),
            pl.BlockSpec((RB, 1), lambda i: (i, 0)),
            pl.BlockSpec((1, D), lambda i: (0, 0)),
        ],
        out_specs=pl.BlockSpec((RB, D), lambda i: (i, 0)),
        out_shape=jax.ShapeDtypeStruct((N, D), jnp.float32),
    )(P, y, dinv, b)


def _pad_y(y):
    return jnp.concatenate([y, jnp.zeros((YPAD - N, D), jnp.float32)], axis=0)


def kernel(x, edge_index, W1, b1, W2, b2, W3, b3, g1, be1, g2, be2):
    src = edge_index[0].astype(jnp.int32)
    dst = edge_index[1].astype(jnp.int32)
    pad = jnp.full((EP - E,), N, jnp.int32)
    srcr = jnp.concatenate([src, pad]).reshape(EP // 128, 128)
    dstr = jnp.concatenate([dst, pad]).reshape(EP // 128, 128)
    z128 = jnp.zeros((ACC_ROWS // NS, D), jnp.float32)
    b1r, b2r, b3r = b1.reshape(1, D), b2.reshape(1, D), b3.reshape(1, D)
    g1r, be1r = g1.reshape(1, D), be1.reshape(1, D)
    g2r, be2r = g2.reshape(1, D), be2.reshape(1, D)

    ones128 = jnp.ones((128, D), jnp.float32)
    degp = _sc_deg(dstr, ones128, z128)
    dinv = _dinv(degp)

    y1 = _mm_scale(x, W1, dinv)
    P1 = _sc_agg(_pad_y(y1), srcr, dstr, z128)
    o1, ps1, pq1 = _combine_stats(P1, y1, dinv, b1r)
    y2 = _bn_relu_mm(o1, ps1, pq1, g1r, be1r, W2, dinv)
    P2 = _sc_agg(_pad_y(y2), srcr, dstr, z128)
    o2, ps2, pq2 = _combine_stats(P2, y2, dinv, b2r)
    y3 = _bn_relu_mm(o2, ps2, pq2, g2r, be2r, W3, dinv)
    P3 = _sc_agg(_pad_y(y3), srcr, dstr, z128)
    return _final(P3, y3, dinv, b3r)


# speedup vs baseline: 9.7204x; 1.5598x over previous
import functools

import jax
import jax.numpy as jnp
from jax import lax
from jax.experimental import pallas as pl
from jax.experimental.pallas import tpu as pltpu
from jax.experimental.pallas import tpu_sc as plsc

N = 10000
E = 320000
D = 128
NC, NS = 2, 16
NW = NC * NS
EB = 1024
JB = EB // 128
SUB = 2
WAVES = JB // SUB
BPT = 10
EP = NW * BPT * EB
OROWS = 632
ACC_ROWS = 10240
YPAD = N + 8
RB = 1000
GRID = N // RB
EPS = 1e-5

_mesh = plsc.VectorSubcoreMesh(core_axis_name="c", subcore_axis_name="s")


W16 = 16
WPS = EB // W16
NBUF = 3
HALF_N = N // NC
ACC2 = 5120
TRASH = 120
TPB = NC * BPT
SROWS = 312


@functools.partial(
    pl.kernel,
    out_type=jax.ShapeDtypeStruct((N, D), jnp.float32),
    mesh=_mesh,
    scratch_types=[
        pltpu.VMEM((JB, 128), jnp.int32),
        pltpu.VMEM((JB, 128), jnp.int32),
        pltpu.VMEM((W16, D), jnp.float32),
        pltpu.VMEM((W16, D), jnp.float32),
        pltpu.VMEM((W16, D), jnp.float32),
        pltpu.VMEM_SHARED((N, D), jnp.float32),
        pltpu.VMEM_SHARED((ACC2, D), jnp.float32),
        pltpu.SemaphoreType.DMA,
        pltpu.SemaphoreType.DMA,
    ],
)
def _sc_agg(y_hbm, src_hbm, dstl_hbm, zeros_hbm, out_hbm,
            src_v, dst_v, b0, b1, b2, ysp, acc, sem_g, sem_s):
    cid = lax.axis_index("c")
    sid = lax.axis_index("s")
    bufs = (b0, b1, b2)
    pltpu.sync_copy(zeros_hbm, acc.at[pl.ds(sid * (ACC2 // NS), ACC2 // NS)])

    @pl.when(sid < NS - 1)
    def _():
        pltpu.sync_copy(y_hbm.at[pl.ds(sid * OROWS, OROWS)],
                        ysp.at[pl.ds(sid * OROWS, OROWS)])

    @pl.when(sid == NS - 1)
    def _():
        pltpu.sync_copy(y_hbm.at[pl.ds((NS - 1) * OROWS, N - (NS - 1) * OROWS)],
                        ysp.at[pl.ds((NS - 1) * OROWS, N - (NS - 1) * OROWS)])

    plsc.subcore_barrier()

    def _idx(ref, j):
        return ref[j // 8, pl.ds((j % 8) * W16, W16)]

    def body(b, carry):
        g = sid * TPB + b
        pltpu.sync_copy(src_hbm.at[pl.ds(g * JB, JB)], src_v)
        pltpu.sync_copy(dstl_hbm.at[cid, pl.ds(g * JB, JB)], dst_v)

        def group(base):
            gat = [pltpu.async_copy(ysp.at[_idx(src_v, base + t)],
                                    bufs[t], sem_g)
                   for t in range(NBUF)]
            sca = []
            for t in range(NBUF):
                gat[t].wait()
                sca.append(pltpu.async_copy(bufs[t],
                                            acc.at[_idx(dst_v, base + t)],
                                            sem_s, add=True))
            for s in sca:
                s.wait()

        def inner(i, c2):
            group(i * NBUF)
            return c2

        lax.fori_loop(0, WPS // NBUF, inner, 0)
        for j in range(WPS - WPS % NBUF, WPS):
            pltpu.async_copy(ysp.at[_idx(src_v, j)], bufs[0], sem_g).wait()
            pltpu.async_copy(bufs[0], acc.at[_idx(dst_v, j)],
                             sem_s, add=True).wait()
        return carry

    lax.fori_loop(0, TPB, body, 0)
    plsc.subcore_barrier()

    @pl.when(sid < NS - 1)
    def _():
        pltpu.sync_copy(acc.at[pl.ds(sid * SROWS, SROWS)],
                        out_hbm.at[pl.ds(cid * HALF_N + sid * SROWS, SROWS)])

    @pl.when(sid == NS - 1)
    def _():
        pltpu.sync_copy(
            acc.at[pl.ds((NS - 1) * SROWS, HALF_N - (NS - 1) * SROWS)],
            out_hbm.at[pl.ds(cid * HALF_N + (NS - 1) * SROWS,
                             HALF_N - (NS - 1) * SROWS)])


@functools.partial(
    pl.kernel,
    out_type=jax.ShapeDtypeStruct((NC, N, D), jnp.float32),
    mesh=_mesh,
    scratch_types=[
        pltpu.VMEM((JB, 128), jnp.int32),
        pltpu.VMEM((128, D), jnp.float32),
        pltpu.VMEM_SHARED((ACC_ROWS, D), jnp.float32),
        pltpu.SemaphoreType.DMA,
    ],
)
def _sc_deg(dst_hbm, ones_hbm, zeros_hbm, out_hbm, dst_v, ones_v, acc, sem_s):
    cid = lax.axis_index("c")
    sid = lax.axis_index("s")
    wid = sid * NC + cid
    pltpu.sync_copy(ones_hbm, ones_v)
    pltpu.sync_copy(zeros_hbm, acc.at[pl.ds(sid * (ACC_ROWS // NS), ACC_ROWS // NS)])
    plsc.subcore_barrier()

    def body(b, carry):
        g = wid * BPT + b
        pltpu.sync_copy(dst_hbm.at[pl.ds(g * JB, JB)], dst_v)
        sca = [
            pltpu.async_copy(ones_v, acc.at[dst_v.at[j]], sem_s, add=True)
            for j in range(JB)
        ]
        for s in sca:
            s.wait()
        return carry

    lax.fori_loop(0, BPT, body, 0)
    plsc.subcore_barrier()

    @pl.when(sid < NS - 1)
    def _():
        pltpu.sync_copy(acc.at[pl.ds(sid * OROWS, OROWS)],
                        out_hbm.at[cid, pl.ds(sid * OROWS, OROWS)])

    @pl.when(sid == NS - 1)
    def _():
        pltpu.sync_copy(acc.at[pl.ds((NS - 1) * OROWS, N - (NS - 1) * OROWS)],
                        out_hbm.at[cid, pl.ds((NS - 1) * OROWS, N - (NS - 1) * OROWS)])


def _dinv_body(dp_ref, o_ref):
    deg = 1.0 + dp_ref[0, :, 0:1] + dp_ref[1, :, 0:1]
    o_ref[...] = lax.rsqrt(deg)


def _dinv(degp):
    return pl.pallas_call(
        _dinv_body,
        grid=(GRID,),
        in_specs=[pl.BlockSpec((NC, RB, D), lambda i: (0, i, 0))],
        out_specs=pl.BlockSpec((RB, 1), lambda i: (i, 0)),
        out_shape=jax.ShapeDtypeStruct((N, 1), jnp.float32),
    )(degp)


def _mm_body(z_ref, w_ref, dinv_ref, o_ref):
    h = jnp.dot(z_ref[...], w_ref[...], preferred_element_type=jnp.float32)
    o_ref[...] = h * dinv_ref[...]


def _mm_scale(z, W, dinv):
    return pl.pallas_call(
        _mm_body,
        grid=(GRID,),
        in_specs=[
            pl.BlockSpec((RB, D), lambda i: (i, 0)),
            pl.BlockSpec((D, D), lambda i: (0, 0)),
            pl.BlockSpec((RB, 1), lambda i: (i, 0)),
        ],
        out_specs=pl.BlockSpec((RB, D), lambda i: (i, 0)),
        out_shape=jax.ShapeDtypeStruct((N, D), jnp.float32),
    )(z, W, dinv)


def _comb_body(p_ref, y_ref, dinv_ref, b_ref, o_ref, ps_ref, pq_ref,
               acc_s, acc_q):
    i = pl.program_id(0)
    o = dinv_ref[...] * (p_ref[...] + y_ref[...]) + b_ref[...]
    o_ref[...] = o
    s = jnp.sum(o, axis=0, keepdims=True)
    q = jnp.sum(o * o, axis=0, keepdims=True)

    @pl.when(i == 0)
    def _():
        acc_s[...] = s
        acc_q[...] = q

    @pl.when(i > 0)
    def _():
        acc_s[...] += s
        acc_q[...] += q

    @pl.when(i == GRID - 1)
    def _():
        ps_ref[...] = acc_s[...]
        pq_ref[...] = acc_q[...]


def _combine_stats(P, y, dinv, b):
    return pl.pallas_call(
        _comb_body,
        grid=(GRID,),
        in_specs=[
            pl.BlockSpec((RB, D), lambda i: (i, 0)),
            pl.BlockSpec((RB, D), lambda i: (i, 0)),
            pl.BlockSpec((RB, 1), lambda i: (i, 0)),
            pl.BlockSpec((1, D), lambda i: (0, 0)),
        ],
        out_specs=[
            pl.BlockSpec((RB, D), lambda i: (i, 0)),
            pl.BlockSpec((1, D), lambda i: (0, 0)),
            pl.BlockSpec((1, D), lambda i: (0, 0)),
        ],
        out_shape=[
            jax.ShapeDtypeStruct((N, D), jnp.float32),
            jax.ShapeDtypeStruct((1, D), jnp.float32),
            jax.ShapeDtypeStruct((1, D), jnp.float32),
        ],
        scratch_shapes=[
            pltpu.VMEM((1, D), jnp.float32),
            pltpu.VMEM((1, D), jnp.float32),
        ],
    )(P, y, dinv, b)


def _bn_mm_body(o_ref, ps_ref, pq_ref, g_ref, be_ref, w_ref, dinv_ref, y_ref):
    mu = ps_ref[...] * (1.0 / N)
    var = pq_ref[...] * (1.0 / N) - mu * mu
    scale = lax.rsqrt(var + EPS) * g_ref[...]
    z = jnp.maximum((o_ref[...] - mu) * scale + be_ref[...], 0.0)
    h = jnp.dot(z, w_ref[...], preferred_element_type=jnp.float32)
    y_ref[...] = h * dinv_ref[...]


def _bn_relu_mm(o, ps, pq, g, be, W, dinv):
    return pl.pallas_call(
        _bn_mm_body,
        grid=(GRID,),
        in_specs=[
            pl.BlockSpec((RB, D), lambda i: (i, 0)),
            pl.BlockSpec((1, D), lambda i: (0, 0)),
            pl.BlockSpec((1, D), lambda i: (0, 0)),
            pl.BlockSpec((1, D), lambda i: (0, 0)),
            pl.BlockSpec((1, D), lambda i: (0, 0)),
            pl.BlockSpec((D, D), lambda i: (0, 0)),
            pl.BlockSpec((RB, 1), lambda i: (i, 0)),
        ],
        out_specs=pl.BlockSpec((RB, D), lambda i: (i, 0)),
        out_shape=jax.ShapeDtypeStruct((N, D), jnp.float32),
    )(o, ps, pq, g, be, W, dinv)


def _final_body(p_ref, y_ref, dinv_ref, b_ref, o_ref):
    o = dinv_ref[...] * (p_ref[...] + y_ref[...]) + b_ref[...]
    o_ref[...] = jnp.maximum(o, 0.0)


def _final(P, y, dinv, b):
    return pl.pallas_call(
        _final_body,
        grid=(GRID,),
        in_specs=[
            pl.BlockSpec((RB, D), lambda i: (i, 0)),
            pl.BlockSpec((RB, D), lambda i: (i, 0)),
            pl.BlockSpec((RB, 1), lambda i: (i, 0)),
            pl.BlockSpec((1, D), lambda i: (0, 0)),
        ],
        out_specs=pl.BlockSpec((RB, D), lambda i: (i, 0)),
        out_shape=jax.ShapeDtypeStruct((N, D), jnp.float32),
    )(P, y, dinv, b)


def kernel(x, edge_index, W1, b1, W2, b2, W3, b3, g1, be1, g2, be2):
    src = edge_index[0].astype(jnp.int32)
    dst = edge_index[1].astype(jnp.int32)
    dstr = jnp.concatenate([dst, jnp.full((EP - E,), N, jnp.int32)]
                           ).reshape(EP // 128, 128)
    srcr = jnp.concatenate([src, jnp.zeros((EP - E,), jnp.int32)]
                           ).reshape(EP // 128, 128)
    trash = HALF_N + (jnp.arange(EP, dtype=jnp.int32) % TRASH)
    dstp = jnp.concatenate([dst, jnp.full((EP - E,), -1, jnp.int32)])
    dstl0 = jnp.where((dstp >= 0) & (dstp < HALF_N), dstp, trash)
    dstl1 = jnp.where(dstp >= HALF_N, dstp - HALF_N, trash)
    dstl = jnp.stack([dstl0, dstl1]).reshape(NC, EP // 128, 128)

    z640 = jnp.zeros((ACC_ROWS // NS, D), jnp.float32)
    z320 = jnp.zeros((ACC2 // NS, D), jnp.float32)
    ones128 = jnp.ones((128, D), jnp.float32)
    b1r, b2r, b3r = b1.reshape(1, D), b2.reshape(1, D), b3.reshape(1, D)
    g1r, be1r = g1.reshape(1, D), be1.reshape(1, D)
    g2r, be2r = g2.reshape(1, D), be2.reshape(1, D)

    degp = _sc_deg(dstr, ones128, z640)
    dinv = _dinv(degp)

    y1 = _mm_scale(x, W1, dinv)
    P1 = _sc_agg(y1, srcr, dstl, z320)
    o1, ps1, pq1 = _combine_stats(P1, y1, dinv, b1r)
    y2 = _bn_relu_mm(o1, ps1, pq1, g1r, be1r, W2, dinv)
    P2 = _sc_agg(y2, srcr, dstl, z320)
    o2, ps2, pq2 = _combine_stats(P2, y2, dinv, b2r)
    y3 = _bn_relu_mm(o2, ps2, pq2, g2r, be2r, W3, dinv)
    P3 = _sc_agg(y3, srcr, dstl, z320)
    return _final(P3, y3, dinv, b3r)


# speedup vs baseline: 11.5072x; 1.1838x over previous
import functools

import jax
import jax.numpy as jnp
from jax import lax
from jax.experimental import pallas as pl
from jax.experimental.pallas import tpu as pltpu
from jax.experimental.pallas import tpu_sc as plsc

N = 10000
E = 320000
D = 128
NC, NS = 2, 16
NW = NC * NS
EB = 1024
JB = EB // 128
SUB = 2
WAVES = JB // SUB
BPT = 10
EP = NW * BPT * EB
OROWS = 632
ACC_ROWS = 10240
YPAD = N + 8
RB = 1000
GRID = N // RB
EPS = 1e-5

_mesh = plsc.VectorSubcoreMesh(core_axis_name="c", subcore_axis_name="s")


W16 = 16
WPS = EB // W16
NBUF = 2
GW = 8
HALF_N = N // NC
ACC2 = 5120
TRASH = 120
TPB = NC * BPT
SROWS = 312


@functools.partial(
    pl.kernel,
    out_type=jax.ShapeDtypeStruct((N, D), jnp.float32),
    mesh=_mesh,
    scratch_types=[
        pltpu.VMEM((2, JB, 128), jnp.int32),
        pltpu.VMEM((2, JB, 128), jnp.int32),
        pltpu.VMEM((W16, D), jnp.float32),
        pltpu.VMEM((W16, D), jnp.float32),
        pltpu.VMEM_SHARED((N, D), jnp.float32),
        pltpu.VMEM_SHARED((ACC2, D), jnp.float32),
        pltpu.SemaphoreType.DMA,
        pltpu.SemaphoreType.DMA,
        pltpu.SemaphoreType.DMA,
    ],
)
def _sc_agg(y_hbm, src_hbm, dstl_hbm, zeros_hbm, out_hbm,
            src_v, dst_v, b0, b1, ysp, acc, sem_g, sem_s, sem_i):
    cid = lax.axis_index("c")
    sid = lax.axis_index("s")
    bufs = (b0, b1)
    pltpu.sync_copy(zeros_hbm, acc.at[pl.ds(sid * (ACC2 // NS), ACC2 // NS)])

    g0 = sid * TPB
    pltpu.async_copy(src_hbm.at[pl.ds(g0 * JB, JB)], src_v.at[0], sem_i)
    pltpu.async_copy(dstl_hbm.at[cid, pl.ds(g0 * JB, JB)], dst_v.at[0], sem_i)

    @pl.when(sid < NS - 1)
    def _():
        pltpu.sync_copy(y_hbm.at[pl.ds(sid * OROWS, OROWS)],
                        ysp.at[pl.ds(sid * OROWS, OROWS)])

    @pl.when(sid == NS - 1)
    def _():
        pltpu.sync_copy(y_hbm.at[pl.ds((NS - 1) * OROWS, N - (NS - 1) * OROWS)],
                        ysp.at[pl.ds((NS - 1) * OROWS, N - (NS - 1) * OROWS)])

    plsc.subcore_barrier()

    def superblock(b, par):
        pltpu.make_async_copy(src_hbm.at[pl.ds(0, JB)],
                              src_v.at[par], sem_i).wait()
        pltpu.make_async_copy(src_hbm.at[pl.ds(0, JB)],
                              dst_v.at[par], sem_i).wait()

        @pl.when(b < TPB - 1)
        def _():
            gn = sid * TPB + b + 1
            pltpu.async_copy(src_hbm.at[pl.ds(gn * JB, JB)],
                             src_v.at[1 - par], sem_i)
            pltpu.async_copy(dstl_hbm.at[cid, pl.ds(gn * JB, JB)],
                             dst_v.at[1 - par], sem_i)

        sv = src_v.at[par]
        dv = dst_v.at[par]

        def _gi(ref, gi, j):
            return ref[gi, pl.ds(j * W16, W16)]

        def group_body(gi, c2):
            gat = [None] * GW
            sca = [None] * GW
            for j in range(GW):
                if j >= NBUF:
                    sca[j - NBUF].wait()
                gat[j] = pltpu.async_copy(ysp.at[_gi(sv, gi, j)],
                                          bufs[j % NBUF], sem_g)
                if j >= 1:
                    gat[j - 1].wait()
                    sca[j - 1] = pltpu.async_copy(bufs[(j - 1) % NBUF],
                                                  acc.at[_gi(dv, gi, j - 1)],
                                                  sem_s, add=True)
            gat[GW - 1].wait()
            sca[GW - 1] = pltpu.async_copy(bufs[(GW - 1) % NBUF],
                                           acc.at[_gi(dv, gi, GW - 1)],
                                           sem_s, add=True)
            for j in range(GW - NBUF, GW):
                sca[j].wait()
            return c2

        lax.fori_loop(0, JB, group_body, 0)

    def body(bb, carry):
        superblock(2 * bb, 0)
        superblock(2 * bb + 1, 1)
        return carry

    lax.fori_loop(0, TPB // 2, body, 0)
    plsc.subcore_barrier()

    @pl.when(sid < NS - 1)
    def _():
        pltpu.sync_copy(acc.at[pl.ds(sid * SROWS, SROWS)],
                        out_hbm.at[pl.ds(cid * HALF_N + sid * SROWS, SROWS)])

    @pl.when(sid == NS - 1)
    def _():
        pltpu.sync_copy(
            acc.at[pl.ds((NS - 1) * SROWS, HALF_N - (NS - 1) * SROWS)],
            out_hbm.at[pl.ds(cid * HALF_N + (NS - 1) * SROWS,
                             HALF_N - (NS - 1) * SROWS)])


@functools.partial(
    pl.kernel,
    out_type=jax.ShapeDtypeStruct((NC, N, D), jnp.float32),
    mesh=_mesh,
    scratch_types=[
        pltpu.VMEM((JB, 128), jnp.int32),
        pltpu.VMEM((128, D), jnp.float32),
        pltpu.VMEM_SHARED((ACC_ROWS, D), jnp.float32),
        pltpu.SemaphoreType.DMA,
    ],
)
def _sc_deg(dst_hbm, ones_hbm, zeros_hbm, out_hbm, dst_v, ones_v, acc, sem_s):
    cid = lax.axis_index("c")
    sid = lax.axis_index("s")
    wid = sid * NC + cid
    pltpu.sync_copy(ones_hbm, ones_v)
    pltpu.sync_copy(zeros_hbm, acc.at[pl.ds(sid * (ACC_ROWS // NS), ACC_ROWS // NS)])
    plsc.subcore_barrier()

    def body(b, carry):
        g = wid * BPT + b
        pltpu.sync_copy(dst_hbm.at[pl.ds(g * JB, JB)], dst_v)
        sca = [
            pltpu.async_copy(ones_v, acc.at[dst_v.at[j]], sem_s, add=True)
            for j in range(JB)
        ]
        for s in sca:
            s.wait()
        return carry

    lax.fori_loop(0, BPT, body, 0)
    plsc.subcore_barrier()

    @pl.when(sid < NS - 1)
    def _():
        pltpu.sync_copy(acc.at[pl.ds(sid * OROWS, OROWS)],
                        out_hbm.at[cid, pl.ds(sid * OROWS, OROWS)])

    @pl.when(sid == NS - 1)
    def _():
        pltpu.sync_copy(acc.at[pl.ds((NS - 1) * OROWS, N - (NS - 1) * OROWS)],
                        out_hbm.at[cid, pl.ds((NS - 1) * OROWS, N - (NS - 1) * OROWS)])


def _dinv_body(dp_ref, o_ref):
    deg = 1.0 + dp_ref[0, :, 0:1] + dp_ref[1, :, 0:1]
    o_ref[...] = lax.rsqrt(deg)


def _dinv(degp):
    return pl.pallas_call(
        _dinv_body,
        grid=(GRID,),
        in_specs=[pl.BlockSpec((NC, RB, D), lambda i: (0, i, 0))],
        out_specs=pl.BlockSpec((RB, 1), lambda i: (i, 0)),
        out_shape=jax.ShapeDtypeStruct((N, 1), jnp.float32),
    )(degp)


def _mm_body(z_ref, w_ref, dinv_ref, o_ref):
    h = jnp.dot(z_ref[...], w_ref[...], preferred_element_type=jnp.float32)
    o_ref[...] = h * dinv_ref[...]


def _mm_scale(z, W, dinv):
    return pl.pallas_call(
        _mm_body,
        grid=(GRID,),
        in_specs=[
            pl.BlockSpec((RB, D), lambda i: (i, 0)),
            pl.BlockSpec((D, D), lambda i: (0, 0)),
            pl.BlockSpec((RB, 1), lambda i: (i, 0)),
        ],
        out_specs=pl.BlockSpec((RB, D), lambda i: (i, 0)),
        out_shape=jax.ShapeDtypeStruct((N, D), jnp.float32),
    )(z, W, dinv)


def _comb_body(p_ref, y_ref, dinv_ref, b_ref, o_ref, ps_ref, pq_ref,
               acc_s, acc_q):
    i = pl.program_id(0)
    o = dinv_ref[...] * (p_ref[...] + y_ref[...]) + b_ref[...]
    o_ref[...] = o
    s = jnp.sum(o, axis=0, keepdims=True)
    q = jnp.sum(o * o, axis=0, keepdims=True)

    @pl.when(i == 0)
    def _():
        acc_s[...] = s
        acc_q[...] = q

    @pl.when(i > 0)
    def _():
        acc_s[...] += s
        acc_q[...] += q

    @pl.when(i == GRID - 1)
    def _():
        ps_ref[...] = acc_s[...]
        pq_ref[...] = acc_q[...]


def _combine_stats(P, y, dinv, b):
    return pl.pallas_call(
        _comb_body,
        grid=(GRID,),
        in_specs=[
            pl.BlockSpec((RB, D), lambda i: (i, 0)),
            pl.BlockSpec((RB, D), lambda i: (i, 0)),
            pl.BlockSpec((RB, 1), lambda i: (i, 0)),
            pl.BlockSpec((1, D), lambda i: (0, 0)),
        ],
        out_specs=[
            pl.BlockSpec((RB, D), lambda i: (i, 0)),
            pl.BlockSpec((1, D), lambda i: (0, 0)),
            pl.BlockSpec((1, D), lambda i: (0, 0)),
        ],
        out_shape=[
            jax.ShapeDtypeStruct((N, D), jnp.float32),
            jax.ShapeDtypeStruct((1, D), jnp.float32),
            jax.ShapeDtypeStruct((1, D), jnp.float32),
        ],
        scratch_shapes=[
            pltpu.VMEM((1, D), jnp.float32),
            pltpu.VMEM((1, D), jnp.float32),
        ],
    )(P, y, dinv, b)


def _bn_mm_body(o_ref, ps_ref, pq_ref, g_ref, be_ref, w_ref, dinv_ref, y_ref):
    mu = ps_ref[...] * (1.0 / N)
    var = pq_ref[...] * (1.0 / N) - mu * mu
    scale = lax.rsqrt(var + EPS) * g_ref[...]
    z = jnp.maximum((o_ref[...] - mu) * scale + be_ref[...], 0.0)
    h = jnp.dot(z, w_ref[...], preferred_element_type=jnp.float32)
    y_ref[...] = h * dinv_ref[...]


def _bn_relu_mm(o, ps, pq, g, be, W, dinv):
    return pl.pallas_call(
        _bn_mm_body,
        grid=(GRID,),
        in_specs=[
            pl.BlockSpec((RB, D), lambda i: (i, 0)),
            pl.BlockSpec((1, D), lambda i: (0, 0)),
            pl.BlockSpec((1, D), lambda i: (0, 0)),
            pl.BlockSpec((1, D), lambda i: (0, 0)),
            pl.BlockSpec((1, D), lambda i: (0, 0)),
            pl.BlockSpec((D, D), lambda i: (0, 0)),
            pl.BlockSpec((RB, 1), lambda i: (i, 0)),
        ],
        out_specs=pl.BlockSpec((RB, D), lambda i: (i, 0)),
        out_shape=jax.ShapeDtypeStruct((N, D), jnp.float32),
    )(o, ps, pq, g, be, W, dinv)


def _final_body(p_ref, y_ref, dinv_ref, b_ref, o_ref):
    o = dinv_ref[...] * (p_ref[...] + y_ref[...]) + b_ref[...]
    o_ref[...] = jnp.maximum(o, 0.0)


def _final(P, y, dinv, b):
    return pl.pallas_call(
        _final_body,
        grid=(GRID,),
        in_specs=[
            pl.BlockSpec((RB, D), lambda i: (i, 0)),
            pl.BlockSpec((RB, D), lambda i: (i, 0)),
            pl.BlockSpec((RB, 1), lambda i: (i, 0)),
            pl.BlockSpec((1, D), lambda i: (0, 0)),
        ],
        out_specs=pl.BlockSpec((RB, D), lambda i: (i, 0)),
        out_shape=jax.ShapeDtypeStruct((N, D), jnp.float32),
    )(P, y, dinv, b)


def kernel(x, edge_index, W1, b1, W2, b2, W3, b3, g1, be1, g2, be2):
    src = edge_index[0].astype(jnp.int32)
    dst = edge_index[1].astype(jnp.int32)
    dstr = jnp.concatenate([dst, jnp.full((EP - E,), N, jnp.int32)]
                           ).reshape(EP // 128, 128)
    srcr = jnp.concatenate([src, jnp.zeros((EP - E,), jnp.int32)]
                           ).reshape(EP // 128, 128)
    trash = HALF_N + (jnp.arange(EP, dtype=jnp.int32) % TRASH)
    dstp = jnp.concatenate([dst, jnp.full((EP - E,), -1, jnp.int32)])
    dstl0 = jnp.where((dstp >= 0) & (dstp < HALF_N), dstp, trash)
    dstl1 = jnp.where(dstp >= HALF_N, dstp - HALF_N, trash)
    dstl = jnp.stack([dstl0, dstl1]).reshape(NC, EP // 128, 128)

    z640 = jnp.zeros((ACC_ROWS // NS, D), jnp.float32)
    z320 = jnp.zeros((ACC2 // NS, D), jnp.float32)
    ones128 = jnp.ones((128, D), jnp.float32)
    b1r, b2r, b3r = b1.reshape(1, D), b2.reshape(1, D), b3.reshape(1, D)
    g1r, be1r = g1.reshape(1, D), be1.reshape(1, D)
    g2r, be2r = g2.reshape(1, D), be2.reshape(1, D)

    degp = _sc_deg(dstr, ones128, z640)
    dinv = _dinv(degp)

    y1 = _mm_scale(x, W1, dinv)
    P1 = _sc_agg(y1, srcr, dstl, z320)
    o1, ps1, pq1 = _combine_stats(P1, y1, dinv, b1r)
    y2 = _bn_relu_mm(o1, ps1, pq1, g1r, be1r, W2, dinv)
    P2 = _sc_agg(y2, srcr, dstl, z320)
    o2, ps2, pq2 = _combine_stats(P2, y2, dinv, b2r)
    y3 = _bn_relu_mm(o2, ps2, pq2, g2r, be2r, W3, dinv)
    P3 = _sc_agg(y3, srcr, dstl, z320)
    return _final(P3, y3, dinv, b3r)


# speedup vs baseline: 11.5601x; 1.0046x over previous
import functools

import jax
import jax.numpy as jnp
from jax import lax
from jax.experimental import pallas as pl
from jax.experimental.pallas import tpu as pltpu
from jax.experimental.pallas import tpu_sc as plsc

N = 10000
E = 320000
D = 128
NC, NS = 2, 16
NW = NC * NS
EB = 1024
JB = EB // 128
SUB = 2
WAVES = JB // SUB
BPT = 10
EP = NW * BPT * EB
OROWS = 632
ACC_ROWS = 10240
YPAD = N + 8
RB = 1000
GRID = N // RB
EPS = 1e-5

_mesh = plsc.VectorSubcoreMesh(core_axis_name="c", subcore_axis_name="s")


W16 = 16
WPS = EB // W16
NBUF = 2
GW = 8
HALF_N = N // NC
ACC2 = 5120
TRASH = 120
TPB = NC * BPT
SROWS = 312


@functools.partial(
    pl.kernel,
    out_type=jax.ShapeDtypeStruct((N, D), jnp.float32),
    mesh=_mesh,
    scratch_types=[
        pltpu.VMEM((2, JB, 128), jnp.int32),
        pltpu.VMEM((2, JB, 128), jnp.int32),
        pltpu.VMEM((W16, D), jnp.float32),
        pltpu.VMEM((W16, D), jnp.float32),
        pltpu.VMEM_SHARED((N, D), jnp.float32),
        pltpu.VMEM_SHARED((ACC2, D), jnp.float32),
        pltpu.SemaphoreType.DMA,
        pltpu.SemaphoreType.DMA,
        pltpu.SemaphoreType.DMA,
    ],
)
def _sc_agg(y_hbm, src_hbm, dstl_hbm, zeros_hbm, out_hbm,
            src_v, dst_v, b0, b1, ysp, acc, sem_g, sem_s, sem_i):
    cid = lax.axis_index("c")
    sid = lax.axis_index("s")
    bufs = (b0, b1)
    pltpu.sync_copy(zeros_hbm, acc.at[pl.ds(sid * (ACC2 // NS), ACC2 // NS)])

    g0 = sid * TPB
    pltpu.async_copy(src_hbm.at[pl.ds(g0 * JB, JB)], src_v.at[0], sem_i)
    pltpu.async_copy(dstl_hbm.at[cid, pl.ds(g0 * JB, JB)], dst_v.at[0], sem_i)

    @pl.when(sid < NS - 1)
    def _():
        pltpu.sync_copy(y_hbm.at[pl.ds(sid * OROWS, OROWS)],
                        ysp.at[pl.ds(sid * OROWS, OROWS)])

    @pl.when(sid == NS - 1)
    def _():
        pltpu.sync_copy(y_hbm.at[pl.ds((NS - 1) * OROWS, N - (NS - 1) * OROWS)],
                        ysp.at[pl.ds((NS - 1) * OROWS, N - (NS - 1) * OROWS)])

    plsc.subcore_barrier()

    def superblock(b, par):
        pltpu.make_async_copy(src_hbm.at[pl.ds(0, JB)],
                              src_v.at[par], sem_i).wait()
        pltpu.make_async_copy(src_hbm.at[pl.ds(0, JB)],
                              dst_v.at[par], sem_i).wait()

        @pl.when(b < TPB - 1)
        def _():
            gn = sid * TPB + b + 1
            pltpu.async_copy(src_hbm.at[pl.ds(gn * JB, JB)],
                             src_v.at[1 - par], sem_i)
            pltpu.async_copy(dstl_hbm.at[cid, pl.ds(gn * JB, JB)],
                             dst_v.at[1 - par], sem_i)

        sv = src_v.at[par]
        dv = dst_v.at[par]

        def _gi(ref, gi, j):
            return ref[gi, pl.ds(j * W16, W16)]

        def group_body(gi, c2):
            gat = [None] * GW
            sca = [None] * GW
            for j in range(GW):
                if j >= NBUF:
                    sca[j - NBUF].wait()
                gat[j] = pltpu.async_copy(ysp.at[_gi(sv, gi, j)],
                                          bufs[j % NBUF], sem_g)
                if j >= 1:
                    gat[j - 1].wait()
                    sca[j - 1] = pltpu.async_copy(bufs[(j - 1) % NBUF],
                                                  acc.at[_gi(dv, gi, j - 1)],
                                                  sem_s, add=True)
            gat[GW - 1].wait()
            sca[GW - 1] = pltpu.async_copy(bufs[(GW - 1) % NBUF],
                                           acc.at[_gi(dv, gi, GW - 1)],
                                           sem_s, add=True)
            for j in range(GW - NBUF, GW):
                sca[j].wait()
            return c2

        lax.fori_loop(0, JB, group_body, 0)

    def body(bb, carry):
        superblock(2 * bb, 0)
        superblock(2 * bb + 1, 1)
        return carry

    lax.fori_loop(0, TPB // 2, body, 0)
    plsc.subcore_barrier()

    @pl.when(sid < NS - 1)
    def _():
        pltpu.sync_copy(acc.at[pl.ds(sid * SROWS, SROWS)],
                        out_hbm.at[pl.ds(cid * HALF_N + sid * SROWS, SROWS)])

    @pl.when(sid == NS - 1)
    def _():
        pltpu.sync_copy(
            acc.at[pl.ds((NS - 1) * SROWS, HALF_N - (NS - 1) * SROWS)],
            out_hbm.at[pl.ds(cid * HALF_N + (NS - 1) * SROWS,
                             HALF_N - (NS - 1) * SROWS)])


@functools.partial(
    pl.kernel,
    out_type=jax.ShapeDtypeStruct((NC, N, D), jnp.float32),
    mesh=_mesh,
    scratch_types=[
        pltpu.VMEM((2, JB, 128), jnp.int32),
        pltpu.VMEM((128, D), jnp.float32),
        pltpu.VMEM_SHARED((ACC_ROWS, D), jnp.float32),
        pltpu.SemaphoreType.DMA,
        pltpu.SemaphoreType.DMA,
    ],
)
def _sc_deg(dst_hbm, ones_hbm, zeros_hbm, out_hbm, dst_v, ones_v, acc,
            sem_s, sem_i):
    cid = lax.axis_index("c")
    sid = lax.axis_index("s")
    wid = sid * NC + cid
    pltpu.async_copy(dst_hbm.at[pl.ds(wid * BPT * JB, JB)], dst_v.at[0], sem_i)
    pltpu.sync_copy(ones_hbm, ones_v)
    pltpu.sync_copy(zeros_hbm, acc.at[pl.ds(sid * (ACC_ROWS // NS), ACC_ROWS // NS)])
    plsc.subcore_barrier()

    def superblock(b, par):
        pltpu.make_async_copy(dst_hbm.at[pl.ds(0, JB)],
                              dst_v.at[par], sem_i).wait()

        @pl.when(b < BPT - 1)
        def _():
            gn = wid * BPT + b + 1
            pltpu.async_copy(dst_hbm.at[pl.ds(gn * JB, JB)],
                             dst_v.at[1 - par], sem_i)

        sca = [
            pltpu.async_copy(ones_v, acc.at[dst_v.at[par, j]], sem_s, add=True)
            for j in range(JB)
        ]
        for s in sca:
            s.wait()

    def body(bb, carry):
        superblock(2 * bb, 0)
        superblock(2 * bb + 1, 1)
        return carry

    lax.fori_loop(0, BPT // 2, body, 0)
    plsc.subcore_barrier()

    @pl.when(sid < NS - 1)
    def _():
        pltpu.sync_copy(acc.at[pl.ds(sid * OROWS, OROWS)],
                        out_hbm.at[cid, pl.ds(sid * OROWS, OROWS)])

    @pl.when(sid == NS - 1)
    def _():
        pltpu.sync_copy(acc.at[pl.ds((NS - 1) * OROWS, N - (NS - 1) * OROWS)],
                        out_hbm.at[cid, pl.ds((NS - 1) * OROWS, N - (NS - 1) * OROWS)])


def _dinv_body(dp_ref, o_ref):
    deg = 1.0 + dp_ref[0, :, 0:1] + dp_ref[1, :, 0:1]
    o_ref[...] = lax.rsqrt(deg)


def _dinv(degp):
    return pl.pallas_call(
        _dinv_body,
        grid=(GRID,),
        in_specs=[pl.BlockSpec((NC, RB, D), lambda i: (0, i, 0))],
        out_specs=pl.BlockSpec((RB, 1), lambda i: (i, 0)),
        out_shape=jax.ShapeDtypeStruct((N, 1), jnp.float32),
    )(degp)


def _mm_body(z_ref, w_ref, dinv_ref, o_ref):
    h = jnp.dot(z_ref[...], w_ref[...], preferred_element_type=jnp.float32)
    o_ref[...] = h * dinv_ref[...]


def _mm_scale(z, W, dinv):
    return pl.pallas_call(
        _mm_body,
        grid=(GRID,),
        in_specs=[
            pl.BlockSpec((RB, D), lambda i: (i, 0)),
            pl.BlockSpec((D, D), lambda i: (0, 0)),
            pl.BlockSpec((RB, 1), lambda i: (i, 0)),
        ],
        out_specs=pl.BlockSpec((RB, D), lambda i: (i, 0)),
        out_shape=jax.ShapeDtypeStruct((N, D), jnp.float32),
    )(z, W, dinv)


def _comb_body(p_ref, y_ref, dinv_ref, b_ref, o_ref, ps_ref, pq_ref,
               acc_s, acc_q):
    i = pl.program_id(0)
    o = dinv_ref[...] * (p_ref[...] + y_ref[...]) + b_ref[...]
    o_ref[...] = o
    s = jnp.sum(o, axis=0, keepdims=True)
    q = jnp.sum(o * o, axis=0, keepdims=True)

    @pl.when(i == 0)
    def _():
        acc_s[...] = s
        acc_q[...] = q

    @pl.when(i > 0)
    def _():
        acc_s[...] += s
        acc_q[...] += q

    @pl.when(i == GRID - 1)
    def _():
        ps_ref[...] = acc_s[...]
        pq_ref[...] = acc_q[...]


def _combine_stats(P, y, dinv, b):
    return pl.pallas_call(
        _comb_body,
        grid=(GRID,),
        in_specs=[
            pl.BlockSpec((RB, D), lambda i: (i, 0)),
            pl.BlockSpec((RB, D), lambda i: (i, 0)),
            pl.BlockSpec((RB, 1), lambda i: (i, 0)),
            pl.BlockSpec((1, D), lambda i: (0, 0)),
        ],
        out_specs=[
            pl.BlockSpec((RB, D), lambda i: (i, 0)),
            pl.BlockSpec((1, D), lambda i: (0, 0)),
            pl.BlockSpec((1, D), lambda i: (0, 0)),
        ],
        out_shape=[
            jax.ShapeDtypeStruct((N, D), jnp.float32),
            jax.ShapeDtypeStruct((1, D), jnp.float32),
            jax.ShapeDtypeStruct((1, D), jnp.float32),
        ],
        scratch_shapes=[
            pltpu.VMEM((1, D), jnp.float32),
            pltpu.VMEM((1, D), jnp.float32),
        ],
    )(P, y, dinv, b)


def _bn_mm_body(o_ref, ps_ref, pq_ref, g_ref, be_ref, w_ref, dinv_ref, y_ref):
    mu = ps_ref[...] * (1.0 / N)
    var = pq_ref[...] * (1.0 / N) - mu * mu
    scale = lax.rsqrt(var + EPS) * g_ref[...]
    z = jnp.maximum((o_ref[...] - mu) * scale + be_ref[...], 0.0)
    h = jnp.dot(z, w_ref[...], preferred_element_type=jnp.float32)
    y_ref[...] = h * dinv_ref[...]


def _bn_relu_mm(o, ps, pq, g, be, W, dinv):
    return pl.pallas_call(
        _bn_mm_body,
        grid=(GRID,),
        in_specs=[
            pl.BlockSpec((RB, D), lambda i: (i, 0)),
            pl.BlockSpec((1, D), lambda i: (0, 0)),
            pl.BlockSpec((1, D), lambda i: (0, 0)),
            pl.BlockSpec((1, D), lambda i: (0, 0)),
            pl.BlockSpec((1, D), lambda i: (0, 0)),
            pl.BlockSpec((D, D), lambda i: (0, 0)),
            pl.BlockSpec((RB, 1), lambda i: (i, 0)),
        ],
        out_specs=pl.BlockSpec((RB, D), lambda i: (i, 0)),
        out_shape=jax.ShapeDtypeStruct((N, D), jnp.float32),
    )(o, ps, pq, g, be, W, dinv)


def _final_body(p_ref, y_ref, dinv_ref, b_ref, o_ref):
    o = dinv_ref[...] * (p_ref[...] + y_ref[...]) + b_ref[...]
    o_ref[...] = jnp.maximum(o, 0.0)


def _final(P, y, dinv, b):
    return pl.pallas_call(
        _final_body,
        grid=(GRID,),
        in_specs=[
            pl.BlockSpec((RB, D), lambda i: (i, 0)),
            pl.BlockSpec((RB, D), lambda i: (i, 0)),
            pl.BlockSpec((RB, 1), lambda i: (i, 0)),
            pl.BlockSpec((1, D), lambda i: (0, 0)),
        ],
        out_specs=pl.BlockSpec((RB, D), lambda i: (i, 0)),
        out_shape=jax.ShapeDtypeStruct((N, D), jnp.float32),
    )(P, y, dinv, b)


def kernel(x, edge_index, W1, b1, W2, b2, W3, b3, g1, be1, g2, be2):
    src = edge_index[0].astype(jnp.int32)
    dst = edge_index[1].astype(jnp.int32)
    dstr = jnp.concatenate([dst, jnp.full((EP - E,), N, jnp.int32)]
                           ).reshape(EP // 128, 128)
    srcr = jnp.concatenate([src, jnp.zeros((EP - E,), jnp.int32)]
                           ).reshape(EP // 128, 128)
    trash = HALF_N + (jnp.arange(EP, dtype=jnp.int32) % TRASH)
    dstp = jnp.concatenate([dst, jnp.full((EP - E,), -1, jnp.int32)])
    dstl0 = jnp.where((dstp >= 0) & (dstp < HALF_N), dstp, trash)
    dstl1 = jnp.where(dstp >= HALF_N, dstp - HALF_N, trash)
    dstl = jnp.stack([dstl0, dstl1]).reshape(NC, EP // 128, 128)

    z640 = jnp.zeros((ACC_ROWS // NS, D), jnp.float32)
    z320 = jnp.zeros((ACC2 // NS, D), jnp.float32)
    ones128 = jnp.ones((128, D), jnp.float32)
    b1r, b2r, b3r = b1.reshape(1, D), b2.reshape(1, D), b3.reshape(1, D)
    g1r, be1r = g1.reshape(1, D), be1.reshape(1, D)
    g2r, be2r = g2.reshape(1, D), be2.reshape(1, D)

    degp = _sc_deg(dstr, ones128, z640)
    dinv = _dinv(degp)

    y1 = _mm_scale(x, W1, dinv)
    P1 = _sc_agg(y1, srcr, dstl, z320)
    o1, ps1, pq1 = _combine_stats(P1, y1, dinv, b1r)
    y2 = _bn_relu_mm(o1, ps1, pq1, g1r, be1r, W2, dinv)
    P2 = _sc_agg(y2, srcr, dstl, z320)
    o2, ps2, pq2 = _combine_stats(P2, y2, dinv, b2r)
    y3 = _bn_relu_mm(o2, ps2, pq2, g2r, be2r, W3, dinv)
    P3 = _sc_agg(y3, srcr, dstl, z320)
    return _final(P3, y3, dinv, b3r)


# speedup vs baseline: 13.8784x; 1.2005x over previous
import functools

import jax
import jax.numpy as jnp
from jax import lax
from jax.experimental import pallas as pl
from jax.experimental.pallas import tpu as pltpu
from jax.experimental.pallas import tpu_sc as plsc

N = 10000
E = 320000
D = 128
NC, NS = 2, 16
NW = NC * NS
EB = 1024
JB = EB // 128
SUB = 2
WAVES = JB // SUB
BPT = 10
EP = NW * BPT * EB
OROWS = 632
ACC_ROWS = 10240
YPAD = N + 8
RB = 1000
GRID = N // RB
EPS = 1e-5

_mesh = plsc.VectorSubcoreMesh(core_axis_name="c", subcore_axis_name="s")


W16 = 16
WPS = EB // W16
NBUF = 2
GW = 16
HALF_N = N // NC
ACC2 = 5120
TRASH = 120
TPB = NC * BPT
SROWS = 312
DW = 128


@functools.partial(
    pl.kernel,
    out_type=jax.ShapeDtypeStruct((N, D), jnp.float32),
    mesh=_mesh,
    scratch_types=[
        pltpu.VMEM((2, JB, 128), jnp.int32),
        pltpu.VMEM((2, JB, 128), jnp.int32),
        pltpu.VMEM((W16, D), jnp.float32),
        pltpu.VMEM((W16, D), jnp.float32),
        pltpu.VMEM_SHARED((N, D), jnp.float32),
        pltpu.VMEM_SHARED((ACC2, D), jnp.float32),
        pltpu.SemaphoreType.DMA,
        pltpu.SemaphoreType.DMA,
        pltpu.SemaphoreType.DMA,
    ],
)
def _sc_agg(y_hbm, src_hbm, dstl_hbm, zeros_hbm, out_hbm,
            src_v, dst_v, b0, b1, ysp, acc, sem_g, sem_s, sem_i):
    cid = lax.axis_index("c")
    sid = lax.axis_index("s")
    bufs = (b0, b1)
    pltpu.sync_copy(zeros_hbm, acc.at[pl.ds(sid * (ACC2 // NS), ACC2 // NS)])

    g0 = sid * TPB
    pltpu.async_copy(src_hbm.at[pl.ds(g0 * JB, JB)], src_v.at[0], sem_i)
    pltpu.async_copy(dstl_hbm.at[cid, pl.ds(g0 * JB, JB)], dst_v.at[0], sem_i)

    @pl.when(sid < NS - 1)
    def _():
        pltpu.sync_copy(y_hbm.at[pl.ds(sid * OROWS, OROWS)],
                        ysp.at[pl.ds(sid * OROWS, OROWS)])

    @pl.when(sid == NS - 1)
    def _():
        pltpu.sync_copy(y_hbm.at[pl.ds((NS - 1) * OROWS, N - (NS - 1) * OROWS)],
                        ysp.at[pl.ds((NS - 1) * OROWS, N - (NS - 1) * OROWS)])

    plsc.subcore_barrier()

    def superblock(b, par):
        pltpu.make_async_copy(src_hbm.at[pl.ds(0, JB)],
                              src_v.at[par], sem_i).wait()
        pltpu.make_async_copy(src_hbm.at[pl.ds(0, JB)],
                              dst_v.at[par], sem_i).wait()

        @pl.when(b < TPB - 1)
        def _():
            gn = sid * TPB + b + 1
            pltpu.async_copy(src_hbm.at[pl.ds(gn * JB, JB)],
                             src_v.at[1 - par], sem_i)
            pltpu.async_copy(dstl_hbm.at[cid, pl.ds(gn * JB, JB)],
                             dst_v.at[1 - par], sem_i)

        sv = src_v.at[par]
        dv = dst_v.at[par]

        def _gi(ref, gi, j):
            return ref[gi * 2 + j // 8, pl.ds((j % 8) * W16, W16)]

        def group_body(gi, c2):
            gat = [None] * GW
            sca = [None] * GW
            for j in range(GW):
                if j >= NBUF:
                    sca[j - NBUF].wait()
                gat[j] = pltpu.async_copy(ysp.at[_gi(sv, gi, j)],
                                          bufs[j % NBUF], sem_g)
                if j >= 1:
                    gat[j - 1].wait()
                    sca[j - 1] = pltpu.async_copy(bufs[(j - 1) % NBUF],
                                                  acc.at[_gi(dv, gi, j - 1)],
                                                  sem_s, add=True)
            gat[GW - 1].wait()
            sca[GW - 1] = pltpu.async_copy(bufs[(GW - 1) % NBUF],
                                           acc.at[_gi(dv, gi, GW - 1)],
                                           sem_s, add=True)
            for j in range(GW - NBUF, GW):
                sca[j].wait()
            return c2

        lax.fori_loop(0, JB // 2, group_body, 0)

    def body(bb, carry):
        superblock(2 * bb, 0)
        superblock(2 * bb + 1, 1)
        return carry

    lax.fori_loop(0, TPB // 2, body, 0)
    plsc.subcore_barrier()

    @pl.when(sid < NS - 1)
    def _():
        pltpu.sync_copy(acc.at[pl.ds(sid * SROWS, SROWS)],
                        out_hbm.at[pl.ds(cid * HALF_N + sid * SROWS, SROWS)])

    @pl.when(sid == NS - 1)
    def _():
        pltpu.sync_copy(
            acc.at[pl.ds((NS - 1) * SROWS, HALF_N - (NS - 1) * SROWS)],
            out_hbm.at[pl.ds(cid * HALF_N + (NS - 1) * SROWS,
                             HALF_N - (NS - 1) * SROWS)])


@functools.partial(
    pl.kernel,
    out_type=jax.ShapeDtypeStruct((NC, N, DW), jnp.float32),
    mesh=_mesh,
    scratch_types=[
        pltpu.VMEM((2, JB, 128), jnp.int32),
        pltpu.VMEM((128, DW), jnp.float32),
        pltpu.VMEM_SHARED((ACC_ROWS, DW), jnp.float32),
        pltpu.SemaphoreType.DMA,
        pltpu.SemaphoreType.DMA,
    ],
)
def _sc_deg(dst_hbm, ones_hbm, zeros_hbm, out_hbm, dst_v, ones_v, acc,
            sem_s, sem_i):
    cid = lax.axis_index("c")
    sid = lax.axis_index("s")
    wid = sid * NC + cid
    pltpu.async_copy(dst_hbm.at[pl.ds(wid * BPT * JB, JB)], dst_v.at[0], sem_i)
    pltpu.sync_copy(ones_hbm, ones_v)
    pltpu.sync_copy(zeros_hbm, acc.at[pl.ds(sid * (ACC_ROWS // NS), ACC_ROWS // NS)])
    plsc.subcore_barrier()

    def superblock(b, par):
        pltpu.make_async_copy(dst_hbm.at[pl.ds(0, JB)],
                              dst_v.at[par], sem_i).wait()

        @pl.when(b < BPT - 1)
        def _():
            gn = wid * BPT + b + 1
            pltpu.async_copy(dst_hbm.at[pl.ds(gn * JB, JB)],
                             dst_v.at[1 - par], sem_i)

        sca = [
            pltpu.async_copy(ones_v, acc.at[dst_v.at[par, j]], sem_s, add=True)
            for j in range(JB)
        ]
        for s in sca:
            s.wait()

    def body(bb, carry):
        superblock(2 * bb, 0)
        superblock(2 * bb + 1, 1)
        return carry

    lax.fori_loop(0, BPT // 2, body, 0)
    plsc.subcore_barrier()

    @pl.when(sid < NS - 1)
    def _():
        pltpu.sync_copy(acc.at[pl.ds(sid * OROWS, OROWS)],
                        out_hbm.at[cid, pl.ds(sid * OROWS, OROWS)])

    @pl.when(sid == NS - 1)
    def _():
        pltpu.sync_copy(acc.at[pl.ds((NS - 1) * OROWS, N - (NS - 1) * OROWS)],
                        out_hbm.at[cid, pl.ds((NS - 1) * OROWS, N - (NS - 1) * OROWS)])


def _dinv_body(dp_ref, o_ref):
    deg = 1.0 + dp_ref[0, :, 0:1] + dp_ref[1, :, 0:1]
    o_ref[...] = lax.rsqrt(deg)


def _dinv(degp):
    return pl.pallas_call(
        _dinv_body,
        grid=(GRID,),
        in_specs=[pl.BlockSpec((NC, RB, DW), lambda i: (0, i, 0))],
        out_specs=pl.BlockSpec((RB, 1), lambda i: (i, 0)),
        out_shape=jax.ShapeDtypeStruct((N, 1), jnp.float32),
    )(degp)


def _mm_body(z_ref, w_ref, dinv_ref, o_ref):
    h = jnp.dot(z_ref[...], w_ref[...], preferred_element_type=jnp.float32)
    o_ref[...] = h * dinv_ref[...]


def _mm_scale(z, W, dinv):
    return pl.pallas_call(
        _mm_body,
        grid=(GRID,),
        in_specs=[
            pl.BlockSpec((RB, D), lambda i: (i, 0)),
            pl.BlockSpec((D, D), lambda i: (0, 0)),
            pl.BlockSpec((RB, 1), lambda i: (i, 0)),
        ],
        out_specs=pl.BlockSpec((RB, D), lambda i: (i, 0)),
        out_shape=jax.ShapeDtypeStruct((N, D), jnp.float32),
    )(z, W, dinv)


def _comb_body(p_ref, y_ref, dinv_ref, b_ref, o_ref, ps_ref, pq_ref,
               acc_s, acc_q):
    i = pl.program_id(0)
    o = dinv_ref[...] * (p_ref[...] + y_ref[...]) + b_ref[...]
    o_ref[...] = o
    s = jnp.sum(o, axis=0, keepdims=True)
    q = jnp.sum(o * o, axis=0, keepdims=True)

    @pl.when(i == 0)
    def _():
        acc_s[...] = s
        acc_q[...] = q

    @pl.when(i > 0)
    def _():
        acc_s[...] += s
        acc_q[...] += q

    @pl.when(i == GRID - 1)
    def _():
        ps_ref[...] = acc_s[...]
        pq_ref[...] = acc_q[...]


def _combine_stats(P, y, dinv, b):
    return pl.pallas_call(
        _comb_body,
        grid=(GRID,),
        in_specs=[
            pl.BlockSpec((RB, D), lambda i: (i, 0)),
            pl.BlockSpec((RB, D), lambda i: (i, 0)),
            pl.BlockSpec((RB, 1), lambda i: (i, 0)),
            pl.BlockSpec((1, D), lambda i: (0, 0)),
        ],
        out_specs=[
            pl.BlockSpec((RB, D), lambda i: (i, 0)),
            pl.BlockSpec((1, D), lambda i: (0, 0)),
            pl.BlockSpec((1, D), lambda i: (0, 0)),
        ],
        out_shape=[
            jax.ShapeDtypeStruct((N, D), jnp.float32),
            jax.ShapeDtypeStruct((1, D), jnp.float32),
            jax.ShapeDtypeStruct((1, D), jnp.float32),
        ],
        scratch_shapes=[
            pltpu.VMEM((1, D), jnp.float32),
            pltpu.VMEM((1, D), jnp.float32),
        ],
    )(P, y, dinv, b)


def _bn_mm_body(o_ref, ps_ref, pq_ref, g_ref, be_ref, w_ref, dinv_ref, y_ref):
    mu = ps_ref[...] * (1.0 / N)
    var = pq_ref[...] * (1.0 / N) - mu * mu
    scale = lax.rsqrt(var + EPS) * g_ref[...]
    z = jnp.maximum((o_ref[...] - mu) * scale + be_ref[...], 0.0)
    h = jnp.dot(z, w_ref[...], preferred_element_type=jnp.float32)
    y_ref[...] = h * dinv_ref[...]


def _bn_relu_mm(o, ps, pq, g, be, W, dinv):
    return pl.pallas_call(
        _bn_mm_body,
        grid=(GRID,),
        in_specs=[
            pl.BlockSpec((RB, D), lambda i: (i, 0)),
            pl.BlockSpec((1, D), lambda i: (0, 0)),
            pl.BlockSpec((1, D), lambda i: (0, 0)),
            pl.BlockSpec((1, D), lambda i: (0, 0)),
            pl.BlockSpec((1, D), lambda i: (0, 0)),
            pl.BlockSpec((D, D), lambda i: (0, 0)),
            pl.BlockSpec((RB, 1), lambda i: (i, 0)),
        ],
        out_specs=pl.BlockSpec((RB, D), lambda i: (i, 0)),
        out_shape=jax.ShapeDtypeStruct((N, D), jnp.float32),
    )(o, ps, pq, g, be, W, dinv)


def _final_body(p_ref, y_ref, dinv_ref, b_ref, o_ref):
    o = dinv_ref[...] * (p_ref[...] + y_ref[...]) + b_ref[...]
    o_ref[...] = jnp.maximum(o, 0.0)


def _final(P, y, dinv, b):
    return pl.pallas_call(
        _final_body,
        grid=(GRID,),
        in_specs=[
            pl.BlockSpec((RB, D), lambda i: (i, 0)),
            pl.BlockSpec((RB, D), lambda i: (i, 0)),
            pl.BlockSpec((RB, 1), lambda i: (i, 0)),
            pl.BlockSpec((1, D), lambda i: (0, 0)),
        ],
        out_specs=pl.BlockSpec((RB, D), lambda i: (i, 0)),
        out_shape=jax.ShapeDtypeStruct((N, D), jnp.float32),
    )(P, y, dinv, b)


def kernel(x, edge_index, W1, b1, W2, b2, W3, b3, g1, be1, g2, be2):
    src = edge_index[0].astype(jnp.int32)
    dst = edge_index[1].astype(jnp.int32)
    dstr = jnp.concatenate([dst, jnp.full((EP - E,), N, jnp.int32)]
                           ).reshape(EP // 128, 128)
    srcr = jnp.concatenate([src, jnp.zeros((EP - E,), jnp.int32)]
                           ).reshape(EP // 128, 128)
    trash = HALF_N + (jnp.arange(EP, dtype=jnp.int32) % TRASH)
    dstp = jnp.concatenate([dst, jnp.full((EP - E,), -1, jnp.int32)])
    dstl0 = jnp.where((dstp >= 0) & (dstp < HALF_N), dstp, trash)
    dstl1 = jnp.where(dstp >= HALF_N, dstp - HALF_N, trash)
    dstl = jnp.stack([dstl0, dstl1]).reshape(NC, EP // 128, 128)

    z640 = jnp.zeros((ACC_ROWS // NS, DW), jnp.float32)
    z320 = jnp.zeros((ACC2 // NS, D), jnp.float32)
    ones128 = jnp.ones((128, DW), jnp.float32)
    b1r, b2r, b3r = b1.reshape(1, D), b2.reshape(1, D), b3.reshape(1, D)
    g1r, be1r = g1.reshape(1, D), be1.reshape(1, D)
    g2r, be2r = g2.reshape(1, D), be2.reshape(1, D)

    degp = _sc_deg(dstr, ones128, z640)
    dinv = _dinv(degp)

    y1 = _mm_scale(x, W1, dinv)
    P1 = _sc_agg(y1, srcr, dstl, z320)
    o1, ps1, pq1 = _combine_stats(P1, y1, dinv, b1r)
    y2 = _bn_relu_mm(o1, ps1, pq1, g1r, be1r, W2, dinv)
    P2 = _sc_agg(y2, srcr, dstl, z320)
    o2, ps2, pq2 = _combine_stats(P2, y2, dinv, b2r)
    y3 = _bn_relu_mm(o2, ps2, pq2, g2r, be2r, W3, dinv)
    P3 = _sc_agg(y3, srcr, dstl, z320)
    return _final(P3, y3, dinv, b3r)
